# baseline (speedup 1.0000x reference)
import functools

import jax
import jax.numpy as jnp
from jax import lax
from jax.experimental import pallas as pl
from jax.experimental.pallas import tpu as pltpu
from jax.experimental.pallas import tpu_sc as plsc

_BATCH = 8192
_NUM_CODES = 8192
_CODE_DIM = 32
_BR = 256

_SC_CORES = 2
_SC_SUBCORES = 16
_SC_WORKERS = _SC_CORES * _SC_SUBCORES


_LOG2E = 1.4426950408889634
_LN2 = 0.6931471805599453


def _vq_body(f_ref, cb_ref, idx_ref, stats_ref, cn_ref):
    i = pl.program_id(0)
    f = f_ref[...]
    cb = cb_ref[...]

    @pl.when(i == 0)
    def _norms():
        cn0 = jnp.sum(cb * cb, axis=1)
        cn_ref[...] = cn0[None, :]

    f2 = f + f
    mm2 = lax.dot_general(f2, cb, (((1,), (1,)), ((), ())),
                          preferred_element_type=jnp.float32)
    fn = jnp.sum(f * f, axis=1, keepdims=True)
    d = (fn + cn_ref[...]) - mm2

    m = jnp.min(d, axis=1, keepdims=True)
    dm = d - m
    colf = lax.broadcasted_iota(jnp.int32, (1, _NUM_CODES), 1).astype(jnp.float32)
    idxf = jnp.min(colf + dm * jnp.float32(1.0e38), axis=1, keepdims=True)
    idx_ref[...] = idxf.astype(jnp.int32)

    xl2 = dm * jnp.float32(-_LOG2E)
    e = jnp.exp2(xl2)
    s = jnp.sum(e, axis=1)
    t2 = jnp.sum(e * xl2, axis=1)
    ent_sum = jnp.sum(jnp.log(s) - jnp.float32(_LN2) * t2 / s)
    sq_sum = jnp.sum(m)

    lane = lax.broadcasted_iota(jnp.int32, stats_ref.shape, 1)
    contrib = (jnp.where(lane == 0, sq_sum, 0.0)
               + jnp.where(lane == 1, ent_sum, 0.0))

    @pl.when(i == 0)
    def _init():
        stats_ref[...] = contrib

    @pl.when(i != 0)
    def _acc():
        stats_ref[...] += contrib


def _tc_stage(features, codebook):
    n_blocks = _BATCH // _BR
    return pl.pallas_call(
        _vq_body,
        grid=(n_blocks,),
        in_specs=[
            pl.BlockSpec((_BR, _CODE_DIM), lambda i: (i, 0)),
            pl.BlockSpec((_NUM_CODES, _CODE_DIM), lambda i: (0, 0)),
        ],
        out_specs=[
            pl.BlockSpec((_BR, 1), lambda i: (i, 0)),
            pl.BlockSpec((1, 128), lambda i: (0, 0)),
        ],
        out_shape=[
            jax.ShapeDtypeStruct((_BATCH, 1), jnp.int32),
            jax.ShapeDtypeStruct((1, 128), jnp.float32),
        ],
        scratch_shapes=[
            pltpu.VMEM((1, _NUM_CODES), jnp.float32),
        ],
    )(features, codebook)


_B_PER_W = _BATCH // _SC_WORKERS
_PAD_DIM = 128


@functools.cache
def _sc_gather_kernel():
    mesh = plsc.VectorSubcoreMesh(core_axis_name="c", subcore_axis_name="s")

    @functools.partial(
        pl.kernel,
        mesh=mesh,
        out_type=jax.ShapeDtypeStruct((_BATCH, _PAD_DIM), jnp.float32),
        scratch_types=[
            pltpu.VMEM((_B_PER_W,), jnp.int32),
            pltpu.VMEM((_B_PER_W, _PAD_DIM), jnp.float32),
            pltpu.SemaphoreType.DMA,
        ],
    )
    def _sc_gather(idx_hbm, table_hbm, out_hbm, idx_v, rows_v, sem):
        wid = lax.axis_index("s") * _SC_CORES + lax.axis_index("c")
        base = wid * _B_PER_W
        pltpu.sync_copy(idx_hbm.at[pl.ds(base, _B_PER_W)], idx_v)
        pltpu.async_copy(table_hbm.at[idx_v], rows_v, sem).wait()
        pltpu.sync_copy(rows_v, out_hbm.at[pl.ds(base, _B_PER_W)])

    return _sc_gather


@jax.jit
def kernel(features, codebook):
    idx2d, stats = _tc_stage(features, codebook)
    cb_pad = jnp.pad(codebook, ((0, 0), (0, _PAD_DIM - _CODE_DIM)))
    q = _sc_gather_kernel()(idx2d.reshape(_BATCH), cb_pad)[:, :_CODE_DIM]
    codebook_loss = stats[0, 0] / jnp.float32(_BATCH * _CODE_DIM)
    code_entropy = stats[0, 1] / jnp.float32(_BATCH)
    return (q, codebook_loss, codebook_loss, code_entropy)


# speedup vs baseline: 1.2189x; 1.2189x over previous
import functools

import jax
import jax.numpy as jnp
from jax import lax
from jax.experimental import pallas as pl
from jax.experimental.pallas import tpu as pltpu
from jax.experimental.pallas import tpu_sc as plsc

_BATCH = 8192
_NUM_CODES = 8192
_CODE_DIM = 32
_BR = 512

_SC_CORES = 2
_SC_SUBCORES = 16
_SC_WORKERS = _SC_CORES * _SC_SUBCORES


_LOG2E = 1.4426950408889634
_LN2 = 0.6931471805599453


def _vq_body(f_ref, cb_ref, idx_ref, stats_ref):
    i = pl.program_id(0)
    f = f_ref[...]
    cb = cb_ref[...]

    f2 = f + f
    mm2 = lax.dot_general(f2, cb, (((1,), (1,)), ((), ())),
                          preferred_element_type=jnp.float32)
    fn = jnp.sum(f * f, axis=1, keepdims=True)
    cn = jnp.sum(cb * cb, axis=1)
    d = (fn + cn[None, :]) - mm2

    m = jnp.min(d, axis=1, keepdims=True)
    colf = lax.broadcasted_iota(jnp.int32, (1, _NUM_CODES), 1).astype(jnp.float32)
    idxf = jnp.min(jnp.where(d == m, colf, jnp.float32(3.0e38)),
                   axis=1, keepdims=True)
    idx_ref[...] = idxf.astype(jnp.int32)

    fe = f2 * jnp.float32(_LOG2E)
    mme = lax.dot_general(fe, cb, (((1,), (1,)), ((), ())),
                          preferred_element_type=jnp.float32)
    cne = (cn * jnp.float32(_LOG2E))[None, :]
    xl = mme - cne
    e = jnp.exp2(xl)
    s = jnp.sum(e, axis=1)
    t2 = jnp.sum(e * xl, axis=1)
    ent_sum = jnp.sum(jnp.log(s) - jnp.float32(_LN2) * t2 / s)
    sq_sum = jnp.sum(m)

    lane = lax.broadcasted_iota(jnp.int32, stats_ref.shape, 1)
    contrib = (jnp.where(lane == 0, sq_sum, 0.0)
               + jnp.where(lane == 1, ent_sum, 0.0))

    @pl.when(i == 0)
    def _init():
        stats_ref[...] = contrib

    @pl.when(i != 0)
    def _acc():
        stats_ref[...] += contrib


def _tc_stage(features, codebook):
    n_blocks = _BATCH // _BR
    return pl.pallas_call(
        _vq_body,
        grid=(n_blocks,),
        in_specs=[
            pl.BlockSpec((_BR, _CODE_DIM), lambda i: (i, 0)),
            pl.BlockSpec((_NUM_CODES, _CODE_DIM), lambda i: (0, 0)),
        ],
        out_specs=[
            pl.BlockSpec((_BR, 1), lambda i: (i, 0)),
            pl.BlockSpec((1, 128), lambda i: (0, 0)),
        ],
        out_shape=[
            jax.ShapeDtypeStruct((_BATCH, 1), jnp.int32),
            jax.ShapeDtypeStruct((1, 128), jnp.float32),
        ],
    )(features, codebook)


_B_PER_W = _BATCH // _SC_WORKERS
_PAD_DIM = 128


@functools.cache
def _sc_gather_kernel():
    mesh = plsc.VectorSubcoreMesh(core_axis_name="c", subcore_axis_name="s")

    @functools.partial(
        pl.kernel,
        mesh=mesh,
        out_type=jax.ShapeDtypeStruct((_BATCH, _PAD_DIM), jnp.float32),
        scratch_types=[
            pltpu.VMEM((_B_PER_W,), jnp.int32),
            pltpu.VMEM((_B_PER_W, _PAD_DIM), jnp.float32),
            pltpu.SemaphoreType.DMA,
        ],
    )
    def _sc_gather(idx_hbm, table_hbm, out_hbm, idx_v, rows_v, sem):
        wid = lax.axis_index("s") * _SC_CORES + lax.axis_index("c")
        base = wid * _B_PER_W
        pltpu.sync_copy(idx_hbm.at[pl.ds(base, _B_PER_W)], idx_v)
        pltpu.async_copy(table_hbm.at[idx_v], rows_v, sem).wait()
        pltpu.sync_copy(rows_v, out_hbm.at[pl.ds(base, _B_PER_W)])

    return _sc_gather


@jax.jit
def kernel(features, codebook):
    idx2d, stats = _tc_stage(features, codebook)
    cb_pad = jnp.pad(codebook, ((0, 0), (0, _PAD_DIM - _CODE_DIM)))
    q = _sc_gather_kernel()(idx2d.reshape(_BATCH), cb_pad)[:, :_CODE_DIM]
    codebook_loss = stats[0, 0] / jnp.float32(_BATCH * _CODE_DIM)
    code_entropy = stats[0, 1] / jnp.float32(_BATCH)
    return (q, codebook_loss, codebook_loss, code_entropy)


# speedup vs baseline: 1.3053x; 1.0709x over previous
import functools

import jax
import jax.numpy as jnp
from jax import lax
from jax.experimental import pallas as pl
from jax.experimental.pallas import tpu as pltpu
from jax.experimental.pallas import tpu_sc as plsc

_BATCH = 8192
_NUM_CODES = 8192
_CODE_DIM = 32
_BR = 1024

_SC_CORES = 2
_SC_SUBCORES = 16
_SC_WORKERS = _SC_CORES * _SC_SUBCORES


_LOG2E = 1.4426950408889634
_LN2 = 0.6931471805599453


def _vq_body(f_ref, cb_ref, idx_ref, stats_ref):
    i = pl.program_id(0)
    f = f_ref[...]
    cb = cb_ref[...]

    f2 = f + f
    mm2 = lax.dot_general(f2, cb, (((1,), (1,)), ((), ())),
                          preferred_element_type=jnp.float32)
    fn = jnp.sum(f * f, axis=1, keepdims=True)
    cn = jnp.sum(cb * cb, axis=1)
    d = (fn + cn[None, :]) - mm2

    m = jnp.min(d, axis=1, keepdims=True)
    colf = lax.broadcasted_iota(jnp.int32, (1, _NUM_CODES), 1).astype(jnp.float32)
    idxf = jnp.min(jnp.where(d == m, colf, jnp.float32(3.0e38)),
                   axis=1, keepdims=True)
    idx_ref[...] = idxf.astype(jnp.int32)

    fe = f2 * jnp.float32(_LOG2E)
    mme = lax.dot_general(fe, cb, (((1,), (1,)), ((), ())),
                          preferred_element_type=jnp.float32)
    cne = (cn * jnp.float32(_LOG2E))[None, :]
    xl = mme - cne
    e = jnp.exp2(xl)
    s = jnp.sum(e, axis=1)
    t2 = jnp.sum(e * xl, axis=1)
    ent_sum = jnp.sum(jnp.log(s) - jnp.float32(_LN2) * t2 / s)
    sq_sum = jnp.sum(m)

    lane = lax.broadcasted_iota(jnp.int32, stats_ref.shape, 1)
    contrib = (jnp.where(lane == 0, sq_sum, 0.0)
               + jnp.where(lane == 1, ent_sum, 0.0))

    @pl.when(i == 0)
    def _init():
        stats_ref[...] = contrib

    @pl.when(i != 0)
    def _acc():
        stats_ref[...] += contrib


def _tc_stage(features, codebook):
    n_blocks = _BATCH // _BR
    return pl.pallas_call(
        _vq_body,
        grid=(n_blocks,),
        in_specs=[
            pl.BlockSpec((_BR, _CODE_DIM), lambda i: (i, 0)),
            pl.BlockSpec((_NUM_CODES, _CODE_DIM), lambda i: (0, 0)),
        ],
        out_specs=[
            pl.BlockSpec((_BR, 1), lambda i: (i, 0)),
            pl.BlockSpec((1, 128), lambda i: (0, 0)),
        ],
        out_shape=[
            jax.ShapeDtypeStruct((_BATCH, 1), jnp.int32),
            jax.ShapeDtypeStruct((1, 128), jnp.float32),
        ],
    )(features, codebook)


_B_PER_W = _BATCH // _SC_WORKERS
_PAD_DIM = 128


@functools.cache
def _sc_gather_kernel():
    mesh = plsc.VectorSubcoreMesh(core_axis_name="c", subcore_axis_name="s")

    @functools.partial(
        pl.kernel,
        mesh=mesh,
        out_type=jax.ShapeDtypeStruct((_BATCH, _PAD_DIM), jnp.float32),
        scratch_types=[
            pltpu.VMEM((_B_PER_W,), jnp.int32),
            pltpu.VMEM((_B_PER_W, _PAD_DIM), jnp.float32),
            pltpu.SemaphoreType.DMA,
        ],
    )
    def _sc_gather(idx_hbm, table_hbm, out_hbm, idx_v, rows_v, sem):
        wid = lax.axis_index("s") * _SC_CORES + lax.axis_index("c")
        base = wid * _B_PER_W
        pltpu.sync_copy(idx_hbm.at[pl.ds(base, _B_PER_W)], idx_v)
        pltpu.async_copy(table_hbm.at[idx_v], rows_v, sem).wait()
        pltpu.sync_copy(rows_v, out_hbm.at[pl.ds(base, _B_PER_W)])

    return _sc_gather


@jax.jit
def kernel(features, codebook):
    idx2d, stats = _tc_stage(features, codebook)
    cb_pad = jnp.pad(codebook, ((0, 0), (0, _PAD_DIM - _CODE_DIM)))
    q = _sc_gather_kernel()(idx2d.reshape(_BATCH), cb_pad)[:, :_CODE_DIM]
    codebook_loss = stats[0, 0] / jnp.float32(_BATCH * _CODE_DIM)
    code_entropy = stats[0, 1] / jnp.float32(_BATCH)
    return (q, codebook_loss, codebook_loss, code_entropy)
